# baseline (speedup 1.0000x reference)
import functools

import jax
import jax.numpy as jnp
from jax import lax
from jax.experimental import pallas as pl
from jax.experimental.pallas import tpu as pltpu
from jax.experimental.pallas import tpu_sc as plsc

_EPS = 1e-12
_NC, _NS, _L = 2, 16, 16


def _sc_bias(row24, col_flat, tok):
    H, C = row24.shape
    WC = col_flat.shape[1]
    W = WC // C
    mesh = plsc.VectorSubcoreMesh(core_axis_name="c", subcore_axis_name="s",
                                  num_cores=_NC, num_subcores=_NS)

    @functools.partial(
        pl.kernel,
        out_type=jax.ShapeDtypeStruct((H, WC), jnp.float32),
        mesh=mesh,
        scratch_types=[
            pltpu.VMEM((1, C), jnp.float32),
            pltpu.VMEM((1, C), jnp.float32),
            pltpu.VMEM((1, WC), jnp.float32),
        ],
    )
    def k(row_hbm, col_hbm, tok_hbm, out_hbm, rt_v, tok_v, out_v):
        wid = lax.axis_index("s") * _NC + lax.axis_index("c")

        @pl.when(wid < H)
        def _():
            pltpu.sync_copy(row_hbm.at[pl.ds(wid, 1)], rt_v)
            pltpu.sync_copy(tok_hbm, tok_v)
            pltpu.sync_copy(col_hbm, out_v)
            for kk in range(C // _L):
                sl = pl.ds(kk * _L, _L)
                rt_v[0, sl] = rt_v[0, sl] + tok_v[0, sl]
            for w in range(W):
                for kk in range(C // _L):
                    sl = pl.ds(w * C + kk * _L, _L)
                    slc = pl.ds(kk * _L, _L)
                    out_v[0, sl] = out_v[0, sl] + rt_v[0, slc]
            pltpu.sync_copy(out_v, out_hbm.at[pl.ds(wid, 1)])

    return k(row24, col_flat, tok)


def _tc_body(grid_ref, bias_ref, lnw_ref, lnb_ref, out_ref):
    bb, f = grid_ref.shape[0], grid_ref.shape[1]
    bias = bias_ref[...]
    lnw = lnw_ref[...][None, :, :]
    lnb = lnb_ref[...][None, :, :]
    for s in range(bb):
        x = grid_ref[s]
        m = jnp.sum(x, axis=0) * (1.0 / f)
        e = m + bias
        mu = jnp.mean(e, axis=-1, keepdims=True)
        d = e - mu
        var = jnp.mean(d * d, axis=-1, keepdims=True)
        inv = lax.rsqrt(var + _EPS)
        out_ref[s] = d * inv * lnw + lnb


@functools.partial(jax.jit, static_argnames=("bb",))
def _fused(grid, row_emb, col_emb, tok_emb, ln_w, ln_b, bb=1):
    B, F, H, W, C = grid.shape
    bias = _sc_bias(row_emb[:H], col_emb[:W].reshape(1, W * C),
                    tok_emb.reshape(1, C)).reshape(H, W, C)
    out = pl.pallas_call(
        _tc_body,
        grid=(B // bb,),
        in_specs=[
            pl.BlockSpec((bb, F, H, W, C), lambda b: (b, 0, 0, 0, 0)),
            pl.BlockSpec((H, W, C), lambda b: (0, 0, 0)),
            pl.BlockSpec((1, C), lambda b: (0, 0)),
            pl.BlockSpec((1, C), lambda b: (0, 0)),
        ],
        out_specs=pl.BlockSpec((bb, H, W, C), lambda b: (b, 0, 0, 0)),
        out_shape=jax.ShapeDtypeStruct((B, H, W, C), grid.dtype),
    )(grid, bias, ln_w.reshape(1, C), ln_b.reshape(1, C))
    return out.reshape(B, H * W, C)


def kernel(grid, row_emb, col_emb, tok_emb, ln_w, ln_b):
    return _fused(grid, row_emb, col_emb, tok_emb, ln_w, ln_b)


# speedup vs baseline: 1.4075x; 1.4075x over previous
import functools

import jax
import jax.numpy as jnp
from jax import lax
from jax.experimental import pallas as pl

_EPS = 1e-12


def _body(grid_ref, row_ref, col_ref, tok_ref, lnw_ref, lnb_ref, out_ref):
    f = grid_ref.shape[1]
    row = row_ref[...]
    col = col_ref[...]
    tok = tok_ref[...]
    bias = row[:, None, :] + (col + tok)[None, :, :]
    x = grid_ref[0]
    m = jnp.sum(x, axis=0) * (1.0 / f)
    e = m + bias
    mu = jnp.mean(e, axis=-1, keepdims=True)
    d = e - mu
    var = jnp.mean(d * d, axis=-1, keepdims=True)
    inv = lax.rsqrt(var + _EPS)
    out_ref[0] = (d * inv * lnw_ref[...][None, :, :]
                  + lnb_ref[...][None, :, :])


@jax.jit
def _fused(grid, row_emb, col_emb, tok_emb, ln_w, ln_b):
    B, F, H, W, C = grid.shape
    out = pl.pallas_call(
        _body,
        grid=(B,),
        in_specs=[
            pl.BlockSpec((1, F, H, W, C), lambda b: (b, 0, 0, 0, 0)),
            pl.BlockSpec((H, C), lambda b: (0, 0)),
            pl.BlockSpec((W, C), lambda b: (0, 0)),
            pl.BlockSpec((1, C), lambda b: (0, 0)),
            pl.BlockSpec((1, C), lambda b: (0, 0)),
            pl.BlockSpec((1, C), lambda b: (0, 0)),
        ],
        out_specs=pl.BlockSpec((1, H, W, C), lambda b: (b, 0, 0, 0)),
        out_shape=jax.ShapeDtypeStruct((B, H, W, C), grid.dtype),
    )(grid, row_emb[:H], col_emb[:W], tok_emb.reshape(1, C),
      ln_w.reshape(1, C), ln_b.reshape(1, C))
    return out.reshape(B, H * W, C)


def kernel(grid, row_emb, col_emb, tok_emb, ln_w, ln_b):
    return _fused(grid, row_emb, col_emb, tok_emb, ln_w, ln_b)
